# Initial kernel scaffold; baseline (speedup 1.0000x reference)
import jax
import jax.numpy as jnp
from jax.experimental import pallas as pl


def kernel(x, edge_index, params):
    raise NotImplementedError("write your pallas kernel here")


# speedup vs baseline: 13.9024x; 13.9024x over previous
import functools

import jax
import jax.numpy as jnp
from jax import lax
from jax.experimental import pallas as pl
from jax.experimental.pallas import tpu as pltpu
from jax.experimental.pallas import tpu_sc as plsc

N_NODES = 10000
N_EDGES = 320000
HID = 128
OUT_CH = 40
ALPHA = 0.5
GW = 0.8
EPS = 1e-5
IBN = 1.0 / (1.0 + EPS) ** 0.5

BLK = 2000
GRID = N_NODES // BLK
NC, NS = 2, 16
NW = NC * NS
EPT = N_EDGES // NW
K = 80
NCH = EPT // K
RPT = N_NODES // NS


def _a_body(x_ref, wt_ref, bt_ref, g0g_ref, g0b_ref, wq_ref, bq_ref,
            wk_ref, bk_ref, wv_ref, bv_ref, wg_ref, bg_ref,
            layer0_ref, q_ref, v_ref, g0_ref, kv_ref, ksum_ref, sq_ref,
            sk_ref):
    i = pl.program_id(0)
    x = x_ref[...]
    h = jnp.dot(x, wt_ref[...], preferred_element_type=jnp.float32) + bt_ref[...]
    mu = jnp.mean(h, axis=1, keepdims=True)
    var = jnp.mean((h - mu) ** 2, axis=1, keepdims=True)
    h = g0g_ref[...] * (h - mu) * lax.rsqrt(var + EPS) + g0b_ref[...]
    h = jnp.maximum(h, 0.0)
    layer0_ref[...] = h
    q = jnp.dot(h, wq_ref[...], preferred_element_type=jnp.float32) + bq_ref[...]
    k = jnp.dot(h, wk_ref[...], preferred_element_type=jnp.float32) + bk_ref[...]
    v = jnp.dot(h, wv_ref[...], preferred_element_type=jnp.float32) + bv_ref[...]
    q_ref[...] = q
    v_ref[...] = v
    g = jnp.dot(x, wg_ref[...], preferred_element_type=jnp.float32) + bg_ref[...]
    g0_ref[...] = jnp.maximum(g * IBN, 0.0)
    kv = lax.dot_general(k, v, (((0,), (0,)), ((), ())),
                         preferred_element_type=jnp.float32)
    ks = jnp.sum(k, axis=0, keepdims=True)
    sq = jnp.sum(q * q)
    sk = jnp.sum(k * k)

    @pl.when(i == 0)
    def _():
        kv_ref[...] = kv
        ksum_ref[...] = ks
        sq_ref[...] = jnp.full((1, HID), sq, jnp.float32)
        sk_ref[...] = jnp.full((1, HID), sk, jnp.float32)

    @pl.when(i != 0)
    def _():
        kv_ref[...] += kv
        ksum_ref[...] += ks
        sq_ref[...] += sq
        sk_ref[...] += sk


def _stage_a(x, p):
    row = lambda i: (i, 0)
    acc = lambda i: (0, 0)
    outs = jax.ShapeDtypeStruct((N_NODES, HID), jnp.float32)
    return pl.pallas_call(
        _a_body,
        grid=(GRID,),
        in_specs=[pl.BlockSpec((BLK, HID), row)] + [pl.BlockSpec(w.shape, acc)
                                                   for w in p],
        out_specs=[pl.BlockSpec((BLK, HID), row)] * 4 + [
            pl.BlockSpec((HID, HID), acc),
            pl.BlockSpec((1, HID), acc),
            pl.BlockSpec((1, HID), acc),
            pl.BlockSpec((1, HID), acc),
        ],
        out_shape=[outs, outs, outs, outs,
                   jax.ShapeDtypeStruct((HID, HID), jnp.float32),
                   jax.ShapeDtypeStruct((1, HID), jnp.float32),
                   jax.ShapeDtypeStruct((1, HID), jnp.float32),
                   jax.ShapeDtypeStruct((1, HID), jnp.float32)],
    )(x, *p)


RS = 640
RSL = N_NODES - 15 * RS
CH2 = 128
CH2L = 80


def _sc_deg(col):
    mesh = plsc.VectorSubcoreMesh(core_axis_name="c", subcore_axis_name="s")

    @functools.partial(
        pl.kernel, mesh=mesh,
        out_type=jax.ShapeDtypeStruct((NC * N_NODES,), jnp.float32),
        scratch_types=[
            pltpu.VMEM((K,), jnp.int32),
            pltpu.VMEM((K,), jnp.float32),
            pltpu.VMEM((RS,), jnp.float32),
            pltpu.VMEM_SHARED((N_NODES,), jnp.float32),
        ],
    )
    def k(col_hbm, out_hbm, idx_v, ones_v, zbuf, deg_sh):
        c = lax.axis_index("c")
        s = lax.axis_index("s")

        def fill(j, _):
            ones_v[pl.ds(j * 16, 16)] = jnp.full((16,), 1.0, jnp.float32)
            return 0
        lax.fori_loop(0, K // 16, fill, 0)

        def zfill(j, _):
            zbuf[pl.ds(j * 16, 16)] = jnp.zeros((16,), jnp.float32)
            return 0
        lax.fori_loop(0, RS // 16, zfill, 0)

        @pl.when(s < 15)
        def _():
            pltpu.sync_copy(zbuf, deg_sh.at[pl.ds(s * RS, RS)])

        @pl.when(s == 15)
        def _():
            pltpu.sync_copy(zbuf.at[pl.ds(0, RSL)],
                            deg_sh.at[pl.ds(15 * RS, RSL)])
        plsc.subcore_barrier()

        ebase = (c * NS + s) * EPT

        def body(ch, _):
            off = pl.multiple_of(ebase + ch * K, 8)
            pltpu.sync_copy(col_hbm.at[pl.ds(off, K)], idx_v)
            pltpu.sync_copy(ones_v, deg_sh.at[idx_v], add=True)
            return 0
        lax.fori_loop(0, NCH, body, 0)
        plsc.subcore_barrier()

        obase = c * N_NODES

        @pl.when(s < 15)
        def _():
            pltpu.sync_copy(deg_sh.at[pl.ds(s * RS, RS)], zbuf)
            pltpu.sync_copy(zbuf, out_hbm.at[pl.ds(obase + s * RS, RS)])

        @pl.when(s == 15)
        def _():
            pltpu.sync_copy(deg_sh.at[pl.ds(15 * RS, RSL)],
                            zbuf.at[pl.ds(0, RSL)])
            pltpu.sync_copy(zbuf.at[pl.ds(0, RSL)],
                            out_hbm.at[pl.ds(obase + 15 * RS, RSL)])

    return k(col)


def _sc_agg(row, col, gp, zeros2d):
    mesh = plsc.VectorSubcoreMesh(core_axis_name="c", subcore_axis_name="s")

    @functools.partial(
        pl.kernel, mesh=mesh,
        out_type=jax.ShapeDtypeStruct((NC, N_NODES, HID), jnp.float32),
        scratch_types=[
            pltpu.VMEM((K,), jnp.int32),
            pltpu.VMEM((K,), jnp.int32),
            pltpu.VMEM((K, HID), jnp.float32),
            pltpu.VMEM((CH2, HID), jnp.float32),
            pltpu.VMEM_SHARED((N_NODES, HID), jnp.float32),
            pltpu.SemaphoreType.DMA,
        ],
    )
    def k(row_hbm, col_hbm, gp_hbm, z_hbm, out_hbm, idxr, idxc, rows, stage,
          agg_sh, sem):
        c = lax.axis_index("c")
        s = lax.axis_index("s")
        pltpu.sync_copy(z_hbm, stage)

        @pl.when(s < 15)
        def _():
            for j in range(RS // CH2):
                pltpu.sync_copy(stage,
                                agg_sh.at[pl.ds(s * RS + j * CH2, CH2)])

        @pl.when(s == 15)
        def _():
            for j in range(RSL // CH2L):
                pltpu.sync_copy(stage.at[pl.ds(0, CH2L)],
                                agg_sh.at[pl.ds(15 * RS + j * CH2L, CH2L)])
        plsc.subcore_barrier()

        ebase = (c * NS + s) * EPT

        def body(ch, _):
            off = pl.multiple_of(ebase + ch * K, 8)
            pltpu.sync_copy(row_hbm.at[pl.ds(off, K)], idxr)
            pltpu.sync_copy(col_hbm.at[pl.ds(off, K)], idxc)
            pltpu.async_copy(gp_hbm.at[idxr], rows, sem).wait()
            pltpu.sync_copy(rows, agg_sh.at[idxc], add=True)
            return 0
        lax.fori_loop(0, NCH, body, 0)
        plsc.subcore_barrier()

        @pl.when(s < 15)
        def _():
            for j in range(RS // CH2):
                pltpu.sync_copy(agg_sh.at[pl.ds(s * RS + j * CH2, CH2)],
                                stage)
                pltpu.sync_copy(stage,
                                out_hbm.at[c, pl.ds(s * RS + j * CH2, CH2)])

        @pl.when(s == 15)
        def _():
            for j in range(RSL // CH2L):
                pltpu.sync_copy(agg_sh.at[pl.ds(15 * RS + j * CH2L, CH2L)],
                                stage.at[pl.ds(0, CH2L)])
                pltpu.sync_copy(stage.at[pl.ds(0, CH2L)],
                                out_hbm.at[c, pl.ds(15 * RS + j * CH2L, CH2L)])

    return k(row, col, gp, zeros2d)


def _b_body(degt_ref, g0_ref, s_ref, gp_ref):
    d = jnp.sum(degt_ref[...], axis=1, keepdims=True)
    s = jnp.where(d > 0.0, lax.rsqrt(jnp.maximum(d, 1e-30)), 0.0)
    s_ref[...] = s
    gp_ref[...] = s * g0_ref[...]


def _stage_b(degt, g0):
    row = lambda i: (i, 0)
    return pl.pallas_call(
        _b_body,
        grid=(GRID,),
        in_specs=[pl.BlockSpec((BLK, NC), row), pl.BlockSpec((BLK, HID), row)],
        out_specs=[pl.BlockSpec((BLK, 1), row), pl.BlockSpec((BLK, HID), row)],
        out_shape=[jax.ShapeDtypeStruct((N_NODES, 1), jnp.float32),
                   jax.ShapeDtypeStruct((N_NODES, HID), jnp.float32)],
    )(degt, g0)


def _e_body(q_ref, v_ref, layer0_ref, kv_ref, ksum_ref, sq_ref, sk_ref,
            g1g_ref, g1b_ref, x1_ref):
    den = jnp.sqrt(sq_ref[0, 0]) * jnp.sqrt(sk_ref[0, 0])
    q = q_ref[...]
    v = v_ref[...]
    num = jnp.dot(q, kv_ref[...], preferred_element_type=jnp.float32) / den \
        + N_NODES * v
    nrm = lax.dot_general(q, ksum_ref[...], (((1,), (1,)), ((), ())),
                          preferred_element_type=jnp.float32) / den + N_NODES
    h = ALPHA * (num / nrm) + (1.0 - ALPHA) * layer0_ref[...]
    mu = jnp.mean(h, axis=1, keepdims=True)
    var = jnp.mean((h - mu) ** 2, axis=1, keepdims=True)
    h = g1g_ref[...] * (h - mu) * lax.rsqrt(var + EPS) + g1b_ref[...]
    x1_ref[...] = jnp.maximum(h, 0.0)


def _stage_e(q, v, layer0, kv, ksum, sq, sk, g1g, g1b):
    row = lambda i: (i, 0)
    acc = lambda i: (0, 0)
    return pl.pallas_call(
        _e_body,
        grid=(GRID,),
        in_specs=[pl.BlockSpec((BLK, HID), row)] * 3 + [
            pl.BlockSpec((HID, HID), acc),
            pl.BlockSpec((1, HID), acc),
            pl.BlockSpec((1, HID), acc),
            pl.BlockSpec((1, HID), acc),
            pl.BlockSpec((1, HID), acc),
            pl.BlockSpec((1, HID), acc),
        ],
        out_specs=pl.BlockSpec((BLK, HID), row),
        out_shape=jax.ShapeDtypeStruct((N_NODES, HID), jnp.float32),
    )(q, v, layer0, kv, ksum, sq, sk, g1g, g1b)


def _c_body(pa_ref, pb_ref, s_ref, g0_ref, x1_ref, wc_ref, bc_ref,
            wf_ref, bf_ref, out_ref):
    agg = s_ref[...] * (pa_ref[...] + pb_ref[...])
    g2 = jnp.dot(agg, wc_ref[...], preferred_element_type=jnp.float32) \
        + bc_ref[...]
    g2 = jnp.maximum(g2 * IBN, 0.0)
    x2 = g2 + g0_ref[...]
    comb = GW * x2 + (1.0 - GW) * x1_ref[...]
    out_ref[...] = jnp.dot(comb, wf_ref[...],
                           preferred_element_type=jnp.float32) + bf_ref[...]


def _stage_c(pa, pb, s, g0, x1, wc, bc, wf, bf):
    row = lambda i: (i, 0)
    acc = lambda i: (0, 0)
    return pl.pallas_call(
        _c_body,
        grid=(GRID,),
        in_specs=[pl.BlockSpec((BLK, HID), row), pl.BlockSpec((BLK, HID), row),
                  pl.BlockSpec((BLK, 1), row), pl.BlockSpec((BLK, HID), row),
                  pl.BlockSpec((BLK, HID), row),
                  pl.BlockSpec((HID, HID), acc), pl.BlockSpec((1, HID), acc),
                  pl.BlockSpec((HID, OUT_CH), acc),
                  pl.BlockSpec((1, OUT_CH), acc)],
        out_specs=pl.BlockSpec((BLK, OUT_CH), row),
        out_shape=jax.ShapeDtypeStruct((N_NODES, OUT_CH), jnp.float32),
    )(pa, pb, s, g0, x1, wc, bc, wf, bf)


def kernel(x, edge_index, params):
    p = params
    wts = [p['t_fc_W'].T, p['t_fc_b'][None, :],
           p['t_ln0_g'][None, :], p['t_ln0_b'][None, :],
           p['Wq'].T, p['bq'][None, :], p['Wk'].T, p['bk'][None, :],
           p['Wv'].T, p['bv'][None, :], p['g_fc_W'].T, p['g_fc_b'][None, :]]
    layer0, q, v, g0, kv, ksum, sq, sk = _stage_a(x, wts)

    erow = edge_index[0]
    ecol = edge_index[1]
    degp = _sc_deg(ecol)
    degt = degp.reshape(NC, N_NODES).T
    s, gp = _stage_b(degt, g0)

    zeros2d = jnp.zeros((CH2, HID), jnp.float32)
    aggp = _sc_agg(erow, ecol, gp, zeros2d)

    x1 = _stage_e(q, v, layer0, kv, ksum, sq, sk,
                  p['t_ln1_g'][None, :], p['t_ln1_b'][None, :])

    return _stage_c(aggp[0], aggp[1], s, g0, x1,
                    p['g_conv_W'].T, p['g_conv_b'][None, :],
                    p['fc_W'].T, p['fc_b'][None, :])


# speedup vs baseline: 20.0178x; 1.4399x over previous
import functools

import jax
import jax.numpy as jnp
from jax import lax
from jax.experimental import pallas as pl
from jax.experimental.pallas import tpu as pltpu
from jax.experimental.pallas import tpu_sc as plsc

N_NODES = 10000
N_EDGES = 320000
HID = 128
OUT_CH = 40
ALPHA = 0.5
GW = 0.8
EPS = 1e-5
IBN = 1.0 / (1.0 + EPS) ** 0.5

BLK = 2000
GRID = N_NODES // BLK
NC, NS = 2, 16
NW = NC * NS
EPT = N_EDGES // NW
KC = 128
NCHM = EPT // KC
TAIL = EPT - NCHM * KC


def _a_body(x_ref, wt_ref, bt_ref, g0g_ref, g0b_ref, wq_ref, bq_ref,
            wk_ref, bk_ref, wv_ref, bv_ref, wg_ref, bg_ref,
            layer0_ref, q_ref, v_ref, g0_ref, kv_ref, ksum_ref, sq_ref,
            sk_ref):
    i = pl.program_id(0)
    x = x_ref[...]
    h = jnp.dot(x, wt_ref[...], preferred_element_type=jnp.float32) + bt_ref[...]
    mu = jnp.mean(h, axis=1, keepdims=True)
    var = jnp.mean((h - mu) ** 2, axis=1, keepdims=True)
    h = g0g_ref[...] * (h - mu) * lax.rsqrt(var + EPS) + g0b_ref[...]
    h = jnp.maximum(h, 0.0)
    layer0_ref[...] = h
    q = jnp.dot(h, wq_ref[...], preferred_element_type=jnp.float32) + bq_ref[...]
    k = jnp.dot(h, wk_ref[...], preferred_element_type=jnp.float32) + bk_ref[...]
    v = jnp.dot(h, wv_ref[...], preferred_element_type=jnp.float32) + bv_ref[...]
    q_ref[...] = q
    v_ref[...] = v
    g = jnp.dot(x, wg_ref[...], preferred_element_type=jnp.float32) + bg_ref[...]
    g0_ref[...] = jnp.maximum(g * IBN, 0.0)
    kv = lax.dot_general(k, v, (((0,), (0,)), ((), ())),
                         preferred_element_type=jnp.float32)
    ks = jnp.sum(k, axis=0, keepdims=True)
    sq = jnp.sum(q * q)
    sk = jnp.sum(k * k)

    @pl.when(i == 0)
    def _():
        kv_ref[...] = kv
        ksum_ref[...] = ks
        sq_ref[...] = jnp.full((1, HID), sq, jnp.float32)
        sk_ref[...] = jnp.full((1, HID), sk, jnp.float32)

    @pl.when(i != 0)
    def _():
        kv_ref[...] += kv
        ksum_ref[...] += ks
        sq_ref[...] += sq
        sk_ref[...] += sk


def _stage_a(x, p):
    row = lambda i: (i, 0)
    acc = lambda i: (0, 0)
    outs = jax.ShapeDtypeStruct((N_NODES, HID), jnp.float32)
    return pl.pallas_call(
        _a_body,
        grid=(GRID,),
        in_specs=[pl.BlockSpec((BLK, HID), row)] + [pl.BlockSpec(w.shape, acc)
                                                   for w in p],
        out_specs=[pl.BlockSpec((BLK, HID), row)] * 4 + [
            pl.BlockSpec((HID, HID), acc),
            pl.BlockSpec((1, HID), acc),
            pl.BlockSpec((1, HID), acc),
            pl.BlockSpec((1, HID), acc),
        ],
        out_shape=[outs, outs, outs, outs,
                   jax.ShapeDtypeStruct((HID, HID), jnp.float32),
                   jax.ShapeDtypeStruct((1, HID), jnp.float32),
                   jax.ShapeDtypeStruct((1, HID), jnp.float32),
                   jax.ShapeDtypeStruct((1, HID), jnp.float32)],
    )(x, *p)


RS = 640
RSL = N_NODES - 15 * RS
CH2 = 64
CH2L = 40


def _sc_deg(col):
    mesh = plsc.VectorSubcoreMesh(core_axis_name="c", subcore_axis_name="s")

    @functools.partial(
        pl.kernel, mesh=mesh,
        out_type=jax.ShapeDtypeStruct((NC * N_NODES,), jnp.float32),
        scratch_types=[
            pltpu.VMEM((2, KC), jnp.int32),
            pltpu.VMEM((TAIL,), jnp.int32),
            pltpu.VMEM((KC,), jnp.float32),
            pltpu.VMEM((RS,), jnp.float32),
            pltpu.VMEM_SHARED((N_NODES,), jnp.float32),
            pltpu.SemaphoreType.DMA,
            pltpu.SemaphoreType.DMA,
        ],
    )
    def k(col_hbm, out_hbm, idx2, idxt, ones_v, zbuf, deg_sh, sd0, sd1):
        c = lax.axis_index("c")
        s = lax.axis_index("s")
        sems = (sd0, sd1)

        def fill(j, _):
            ones_v[pl.ds(j * 16, 16)] = jnp.full((16,), 1.0, jnp.float32)
            return 0
        lax.fori_loop(0, KC // 16, fill, 0)

        def zfill(j, _):
            zbuf[pl.ds(j * 16, 16)] = jnp.zeros((16,), jnp.float32)
            return 0
        lax.fori_loop(0, RS // 16, zfill, 0)

        @pl.when(s < 15)
        def _():
            pltpu.sync_copy(zbuf, deg_sh.at[pl.ds(s * RS, RS)])

        @pl.when(s == 15)
        def _():
            pltpu.sync_copy(zbuf.at[pl.ds(0, RSL)],
                            deg_sh.at[pl.ds(15 * RS, RSL)])
        plsc.subcore_barrier()

        ebase = (c * NS + s) * EPT

        toff = pl.multiple_of(ebase + NCHM * KC, 8)
        pltpu.sync_copy(col_hbm.at[pl.ds(toff, TAIL)], idxt)
        pltpu.sync_copy(ones_v.at[pl.ds(0, TAIL)], deg_sh.at[idxt], add=True)

        def load_idx(ch, b):
            off = pl.multiple_of(ebase + ch * KC, 8)
            pltpu.async_copy(col_hbm.at[pl.ds(off, KC)], idx2.at[b], sems[b])

        def wait_idx(b):
            pltpu.make_async_copy(col_hbm.at[pl.ds(0, KC)], idx2.at[b],
                                  sems[b]).wait()

        load_idx(0, 0)

        def pair(j, _):
            wait_idx(0)
            load_idx(2 * j + 1, 1)
            pltpu.sync_copy(ones_v, deg_sh.at[idx2.at[0]], add=True)
            wait_idx(1)

            @pl.when(j != NCHM // 2 - 1)
            def _():
                load_idx(2 * j + 2, 0)
            pltpu.sync_copy(ones_v, deg_sh.at[idx2.at[1]], add=True)
            return 0
        lax.fori_loop(0, NCHM // 2, pair, 0)
        plsc.subcore_barrier()

        obase = c * N_NODES

        @pl.when(s < 15)
        def _():
            pltpu.sync_copy(deg_sh.at[pl.ds(s * RS, RS)], zbuf)
            pltpu.sync_copy(zbuf, out_hbm.at[pl.ds(obase + s * RS, RS)])

        @pl.when(s == 15)
        def _():
            pltpu.sync_copy(deg_sh.at[pl.ds(15 * RS, RSL)],
                            zbuf.at[pl.ds(0, RSL)])
            pltpu.sync_copy(zbuf.at[pl.ds(0, RSL)],
                            out_hbm.at[pl.ds(obase + 15 * RS, RSL)])

    return k(col)


def _sc_agg(row, col, gp, zeros2d):
    mesh = plsc.VectorSubcoreMesh(core_axis_name="c", subcore_axis_name="s")

    @functools.partial(
        pl.kernel, mesh=mesh,
        out_type=jax.ShapeDtypeStruct((NC, N_NODES, HID), jnp.float32),
        scratch_types=[
            pltpu.VMEM((2, KC), jnp.int32),
            pltpu.VMEM((2, KC), jnp.int32),
            pltpu.VMEM((2, KC, HID), jnp.float32),
            pltpu.VMEM((TAIL,), jnp.int32),
            pltpu.VMEM((TAIL,), jnp.int32),
            pltpu.VMEM((TAIL, HID), jnp.float32),
            pltpu.VMEM((CH2, HID), jnp.float32),
            pltpu.VMEM_SHARED((N_NODES, HID), jnp.float32),
            pltpu.SemaphoreType.DMA,
            pltpu.SemaphoreType.DMA,
        ],
    )
    def k(row_hbm, col_hbm, gp_hbm, z_hbm, out_hbm, idxr, idxc, rows,
          idxrt, idxct, rowst, stage, agg_sh, sg0, sg1):
        c = lax.axis_index("c")
        s = lax.axis_index("s")
        sems = (sg0, sg1)
        pltpu.sync_copy(z_hbm, stage)

        @pl.when(s < 15)
        def _():
            for j in range(RS // CH2):
                pltpu.sync_copy(stage,
                                agg_sh.at[pl.ds(s * RS + j * CH2, CH2)])

        @pl.when(s == 15)
        def _():
            for j in range(RSL // CH2L):
                pltpu.sync_copy(stage.at[pl.ds(0, CH2L)],
                                agg_sh.at[pl.ds(15 * RS + j * CH2L, CH2L)])
        plsc.subcore_barrier()

        ebase = (c * NS + s) * EPT

        toff = pl.multiple_of(ebase + NCHM * KC, 8)
        pltpu.sync_copy(row_hbm.at[pl.ds(toff, TAIL)], idxrt)
        pltpu.sync_copy(col_hbm.at[pl.ds(toff, TAIL)], idxct)
        pltpu.async_copy(gp_hbm.at[idxrt], rowst, sg0).wait()
        pltpu.sync_copy(rowst, agg_sh.at[idxct], add=True)

        def load_idx(ch, b):
            off = pl.multiple_of(ebase + ch * KC, 8)
            pltpu.sync_copy(row_hbm.at[pl.ds(off, KC)], idxr.at[b])
            pltpu.sync_copy(col_hbm.at[pl.ds(off, KC)], idxc.at[b])

        def start_gather(b):
            pltpu.async_copy(gp_hbm.at[idxr.at[b]], rows.at[b], sems[b])

        def wait_gather(b):
            pltpu.make_async_copy(gp_hbm.at[idxr.at[b]], rows.at[b],
                                  sems[b]).wait()

        load_idx(0, 0)
        start_gather(0)

        def pair(j, _):
            wait_gather(0)
            load_idx(2 * j + 1, 1)
            start_gather(1)
            pltpu.sync_copy(rows.at[0], agg_sh.at[idxc.at[0]], add=True)
            wait_gather(1)

            @pl.when(j != NCHM // 2 - 1)
            def _():
                load_idx(2 * j + 2, 0)
                start_gather(0)
            pltpu.sync_copy(rows.at[1], agg_sh.at[idxc.at[1]], add=True)
            return 0
        lax.fori_loop(0, NCHM // 2, pair, 0)
        plsc.subcore_barrier()

        @pl.when(s < 15)
        def _():
            for j in range(RS // CH2):
                pltpu.sync_copy(agg_sh.at[pl.ds(s * RS + j * CH2, CH2)],
                                stage)
                pltpu.sync_copy(stage,
                                out_hbm.at[c, pl.ds(s * RS + j * CH2, CH2)])

        @pl.when(s == 15)
        def _():
            for j in range(RSL // CH2L):
                pltpu.sync_copy(agg_sh.at[pl.ds(15 * RS + j * CH2L, CH2L)],
                                stage.at[pl.ds(0, CH2L)])
                pltpu.sync_copy(stage.at[pl.ds(0, CH2L)],
                                out_hbm.at[c, pl.ds(15 * RS + j * CH2L, CH2L)])

    return k(row, col, gp, zeros2d)


def _b_body(degt_ref, g0_ref, s_ref, gp_ref):
    d = jnp.sum(degt_ref[...], axis=1, keepdims=True)
    s = jnp.where(d > 0.0, lax.rsqrt(jnp.maximum(d, 1e-30)), 0.0)
    s_ref[...] = s
    gp_ref[...] = s * g0_ref[...]


def _stage_b(degt, g0):
    row = lambda i: (i, 0)
    return pl.pallas_call(
        _b_body,
        grid=(GRID,),
        in_specs=[pl.BlockSpec((BLK, NC), row), pl.BlockSpec((BLK, HID), row)],
        out_specs=[pl.BlockSpec((BLK, 1), row), pl.BlockSpec((BLK, HID), row)],
        out_shape=[jax.ShapeDtypeStruct((N_NODES, 1), jnp.float32),
                   jax.ShapeDtypeStruct((N_NODES, HID), jnp.float32)],
    )(degt, g0)


def _e_body(q_ref, v_ref, layer0_ref, kv_ref, ksum_ref, sq_ref, sk_ref,
            g1g_ref, g1b_ref, x1_ref):
    den = jnp.sqrt(sq_ref[0, 0]) * jnp.sqrt(sk_ref[0, 0])
    q = q_ref[...]
    v = v_ref[...]
    num = jnp.dot(q, kv_ref[...], preferred_element_type=jnp.float32) / den \
        + N_NODES * v
    nrm = lax.dot_general(q, ksum_ref[...], (((1,), (1,)), ((), ())),
                          preferred_element_type=jnp.float32) / den + N_NODES
    h = ALPHA * (num / nrm) + (1.0 - ALPHA) * layer0_ref[...]
    mu = jnp.mean(h, axis=1, keepdims=True)
    var = jnp.mean((h - mu) ** 2, axis=1, keepdims=True)
    h = g1g_ref[...] * (h - mu) * lax.rsqrt(var + EPS) + g1b_ref[...]
    x1_ref[...] = jnp.maximum(h, 0.0)


def _stage_e(q, v, layer0, kv, ksum, sq, sk, g1g, g1b):
    row = lambda i: (i, 0)
    acc = lambda i: (0, 0)
    return pl.pallas_call(
        _e_body,
        grid=(GRID,),
        in_specs=[pl.BlockSpec((BLK, HID), row)] * 3 + [
            pl.BlockSpec((HID, HID), acc),
            pl.BlockSpec((1, HID), acc),
            pl.BlockSpec((1, HID), acc),
            pl.BlockSpec((1, HID), acc),
            pl.BlockSpec((1, HID), acc),
            pl.BlockSpec((1, HID), acc),
        ],
        out_specs=pl.BlockSpec((BLK, HID), row),
        out_shape=jax.ShapeDtypeStruct((N_NODES, HID), jnp.float32),
    )(q, v, layer0, kv, ksum, sq, sk, g1g, g1b)


def _c_body(pa_ref, pb_ref, s_ref, g0_ref, x1_ref, wc_ref, bc_ref,
            wf_ref, bf_ref, out_ref):
    agg = s_ref[...] * (pa_ref[...] + pb_ref[...])
    g2 = jnp.dot(agg, wc_ref[...], preferred_element_type=jnp.float32) \
        + bc_ref[...]
    g2 = jnp.maximum(g2 * IBN, 0.0)
    x2 = g2 + g0_ref[...]
    comb = GW * x2 + (1.0 - GW) * x1_ref[...]
    out_ref[...] = jnp.dot(comb, wf_ref[...],
                           preferred_element_type=jnp.float32) + bf_ref[...]


def _stage_c(pa, pb, s, g0, x1, wc, bc, wf, bf):
    row = lambda i: (i, 0)
    acc = lambda i: (0, 0)
    return pl.pallas_call(
        _c_body,
        grid=(GRID,),
        in_specs=[pl.BlockSpec((BLK, HID), row), pl.BlockSpec((BLK, HID), row),
                  pl.BlockSpec((BLK, 1), row), pl.BlockSpec((BLK, HID), row),
                  pl.BlockSpec((BLK, HID), row),
                  pl.BlockSpec((HID, HID), acc), pl.BlockSpec((1, HID), acc),
                  pl.BlockSpec((HID, OUT_CH), acc),
                  pl.BlockSpec((1, OUT_CH), acc)],
        out_specs=pl.BlockSpec((BLK, OUT_CH), row),
        out_shape=jax.ShapeDtypeStruct((N_NODES, OUT_CH), jnp.float32),
    )(pa, pb, s, g0, x1, wc, bc, wf, bf)


def kernel(x, edge_index, params):
    p = params
    wts = [p['t_fc_W'].T, p['t_fc_b'][None, :],
           p['t_ln0_g'][None, :], p['t_ln0_b'][None, :],
           p['Wq'].T, p['bq'][None, :], p['Wk'].T, p['bk'][None, :],
           p['Wv'].T, p['bv'][None, :], p['g_fc_W'].T, p['g_fc_b'][None, :]]
    layer0, q, v, g0, kv, ksum, sq, sk = _stage_a(x, wts)

    erow = edge_index[0]
    ecol = edge_index[1]
    degp = _sc_deg(ecol)
    degt = degp.reshape(NC, N_NODES).T
    s, gp = _stage_b(degt, g0)

    zeros2d = jnp.zeros((CH2, HID), jnp.float32)
    aggp = _sc_agg(erow, ecol, gp, zeros2d)

    x1 = _stage_e(q, v, layer0, kv, ksum, sq, sk,
                  p['t_ln1_g'][None, :], p['t_ln1_b'][None, :])

    return _stage_c(aggp[0], aggp[1], s, g0, x1,
                    p['g_conv_W'].T, p['g_conv_b'][None, :],
                    p['fc_W'].T, p['fc_b'][None, :])


# speedup vs baseline: 25.4865x; 1.2732x over previous
import functools

import jax
import jax.numpy as jnp
from jax import lax
from jax.experimental import pallas as pl
from jax.experimental.pallas import tpu as pltpu
from jax.experimental.pallas import tpu_sc as plsc

N_NODES = 10000
N_EDGES = 320000
HID = 128
OUT_CH = 40
ALPHA = 0.5
GW = 0.8
EPS = 1e-5
IBN = 1.0 / (1.0 + EPS) ** 0.5

BLK = 2000
GRID = N_NODES // BLK
NC, NS = 2, 16
NW = NC * NS
EPT = N_EDGES // NW
KC = 128
NCHM = EPT // KC
TAIL = EPT - NCHM * KC


def _a_body(x_ref, wt_ref, bt_ref, g0g_ref, g0b_ref, wq_ref, bq_ref,
            wk_ref, bk_ref, wv_ref, bv_ref, wg_ref, bg_ref,
            layer0_ref, q_ref, v_ref, g0_ref, kv_ref, ksum_ref, sq_ref,
            sk_ref):
    i = pl.program_id(0)
    x = x_ref[...]
    h = jnp.dot(x, wt_ref[...], preferred_element_type=jnp.float32) + bt_ref[...]
    mu = jnp.mean(h, axis=1, keepdims=True)
    var = jnp.mean((h - mu) ** 2, axis=1, keepdims=True)
    h = g0g_ref[...] * (h - mu) * lax.rsqrt(var + EPS) + g0b_ref[...]
    h = jnp.maximum(h, 0.0)
    layer0_ref[...] = h
    q = jnp.dot(h, wq_ref[...], preferred_element_type=jnp.float32) + bq_ref[...]
    k = jnp.dot(h, wk_ref[...], preferred_element_type=jnp.float32) + bk_ref[...]
    v = jnp.dot(h, wv_ref[...], preferred_element_type=jnp.float32) + bv_ref[...]
    q_ref[...] = q
    v_ref[...] = v
    g = jnp.dot(x, wg_ref[...], preferred_element_type=jnp.float32) + bg_ref[...]
    g0_ref[...] = jnp.maximum(g * IBN, 0.0)
    kv = lax.dot_general(k, v, (((0,), (0,)), ((), ())),
                         preferred_element_type=jnp.float32)
    ks = jnp.sum(k, axis=0, keepdims=True)
    sq = jnp.sum(q * q)
    sk = jnp.sum(k * k)

    @pl.when(i == 0)
    def _():
        kv_ref[...] = kv
        ksum_ref[...] = ks
        sq_ref[...] = jnp.full((1, HID), sq, jnp.float32)
        sk_ref[...] = jnp.full((1, HID), sk, jnp.float32)

    @pl.when(i != 0)
    def _():
        kv_ref[...] += kv
        ksum_ref[...] += ks
        sq_ref[...] += sq
        sk_ref[...] += sk


def _stage_a(x, p):
    row = lambda i: (i, 0)
    acc = lambda i: (0, 0)
    outs = jax.ShapeDtypeStruct((N_NODES, HID), jnp.float32)
    return pl.pallas_call(
        _a_body,
        grid=(GRID,),
        in_specs=[pl.BlockSpec((BLK, HID), row)] + [pl.BlockSpec(w.shape, acc)
                                                   for w in p],
        out_specs=[pl.BlockSpec((BLK, HID), row)] * 4 + [
            pl.BlockSpec((HID, HID), acc),
            pl.BlockSpec((1, HID), acc),
            pl.BlockSpec((1, HID), acc),
            pl.BlockSpec((1, HID), acc),
        ],
        out_shape=[outs, outs, outs, outs,
                   jax.ShapeDtypeStruct((HID, HID), jnp.float32),
                   jax.ShapeDtypeStruct((1, HID), jnp.float32),
                   jax.ShapeDtypeStruct((1, HID), jnp.float32),
                   jax.ShapeDtypeStruct((1, HID), jnp.float32)],
    )(x, *p)


RS = 640
RSL = N_NODES - 15 * RS
CH2 = 64
CH2L = 40


def _sc_deg(col):
    mesh = plsc.VectorSubcoreMesh(core_axis_name="c", subcore_axis_name="s")

    @functools.partial(
        pl.kernel, mesh=mesh,
        out_type=jax.ShapeDtypeStruct((NC * N_NODES,), jnp.float32),
        scratch_types=[
            pltpu.VMEM((2, KC), jnp.int32),
            pltpu.VMEM((TAIL,), jnp.int32),
            pltpu.VMEM((KC,), jnp.float32),
            pltpu.VMEM((RS,), jnp.float32),
            pltpu.VMEM_SHARED((N_NODES,), jnp.float32),
            pltpu.SemaphoreType.DMA,
            pltpu.SemaphoreType.DMA,
        ],
    )
    def k(col_hbm, out_hbm, idx2, idxt, ones_v, zbuf, deg_sh, sd0, sd1):
        c = lax.axis_index("c")
        s = lax.axis_index("s")
        sems = (sd0, sd1)

        def fill(j, _):
            ones_v[pl.ds(j * 16, 16)] = jnp.full((16,), 1.0, jnp.float32)
            return 0
        lax.fori_loop(0, KC // 16, fill, 0)

        def zfill(j, _):
            zbuf[pl.ds(j * 16, 16)] = jnp.zeros((16,), jnp.float32)
            return 0
        lax.fori_loop(0, RS // 16, zfill, 0)

        @pl.when(s < 15)
        def _():
            pltpu.sync_copy(zbuf, deg_sh.at[pl.ds(s * RS, RS)])

        @pl.when(s == 15)
        def _():
            pltpu.sync_copy(zbuf.at[pl.ds(0, RSL)],
                            deg_sh.at[pl.ds(15 * RS, RSL)])
        plsc.subcore_barrier()

        ebase = (c * NS + s) * EPT

        toff = pl.multiple_of(ebase + NCHM * KC, 8)
        pltpu.sync_copy(col_hbm.at[pl.ds(toff, TAIL)], idxt)
        pltpu.sync_copy(ones_v.at[pl.ds(0, TAIL)], deg_sh.at[idxt], add=True)

        def load_idx(ch, b):
            off = pl.multiple_of(ebase + ch * KC, 8)
            pltpu.async_copy(col_hbm.at[pl.ds(off, KC)], idx2.at[b], sems[b])

        def wait_idx(b):
            pltpu.make_async_copy(col_hbm.at[pl.ds(0, KC)], idx2.at[b],
                                  sems[b]).wait()

        load_idx(0, 0)

        def pair(j, _):
            wait_idx(0)
            load_idx(2 * j + 1, 1)
            pltpu.sync_copy(ones_v, deg_sh.at[idx2.at[0]], add=True)
            wait_idx(1)

            @pl.when(j != NCHM // 2 - 1)
            def _():
                load_idx(2 * j + 2, 0)
            pltpu.sync_copy(ones_v, deg_sh.at[idx2.at[1]], add=True)
            return 0
        lax.fori_loop(0, NCHM // 2, pair, 0)
        plsc.subcore_barrier()

        obase = c * N_NODES

        @pl.when(s < 15)
        def _():
            pltpu.sync_copy(deg_sh.at[pl.ds(s * RS, RS)], zbuf)
            pltpu.sync_copy(zbuf, out_hbm.at[pl.ds(obase + s * RS, RS)])

        @pl.when(s == 15)
        def _():
            pltpu.sync_copy(deg_sh.at[pl.ds(15 * RS, RSL)],
                            zbuf.at[pl.ds(0, RSL)])
            pltpu.sync_copy(zbuf.at[pl.ds(0, RSL)],
                            out_hbm.at[pl.ds(obase + 15 * RS, RSL)])

    return k(col)


def _sc_agg(row, col, gp, zeros2d):
    mesh = plsc.VectorSubcoreMesh(core_axis_name="c", subcore_axis_name="s")

    @functools.partial(
        pl.kernel, mesh=mesh,
        out_type=jax.ShapeDtypeStruct((NC, N_NODES, HID), jnp.float32),
        scratch_types=[
            pltpu.VMEM((2, KC), jnp.int32),
            pltpu.VMEM((2, KC), jnp.int32),
            pltpu.VMEM((2, KC, HID), jnp.float32),
            pltpu.VMEM((TAIL,), jnp.int32),
            pltpu.VMEM((TAIL,), jnp.int32),
            pltpu.VMEM((TAIL, HID), jnp.float32),
            pltpu.VMEM((CH2, HID), jnp.float32),
            pltpu.VMEM_SHARED((N_NODES, HID), jnp.float32),
            pltpu.SemaphoreType.DMA,
            pltpu.SemaphoreType.DMA,
            pltpu.SemaphoreType.DMA,
            pltpu.SemaphoreType.DMA,
        ],
    )
    def k(row_hbm, col_hbm, gp_hbm, z_hbm, out_hbm, idxr, idxc, rows,
          idxrt, idxct, rowst, stage, agg_sh, sg0, sg1, si0, si1):
        c = lax.axis_index("c")
        s = lax.axis_index("s")
        sems = (sg0, sg1)
        isems = (si0, si1)
        pltpu.sync_copy(z_hbm, stage)

        @pl.when(s < 15)
        def _():
            for j in range(RS // CH2):
                pltpu.sync_copy(stage,
                                agg_sh.at[pl.ds(s * RS + j * CH2, CH2)])

        @pl.when(s == 15)
        def _():
            for j in range(RSL // CH2L):
                pltpu.sync_copy(stage.at[pl.ds(0, CH2L)],
                                agg_sh.at[pl.ds(15 * RS + j * CH2L, CH2L)])
        plsc.subcore_barrier()

        ebase = (c * NS + s) * EPT

        toff = pl.multiple_of(ebase + NCHM * KC, 8)
        pltpu.sync_copy(row_hbm.at[pl.ds(toff, TAIL)], idxrt)
        pltpu.sync_copy(col_hbm.at[pl.ds(toff, TAIL)], idxct)
        pltpu.async_copy(gp_hbm.at[idxrt], rowst, sg0).wait()
        pltpu.sync_copy(rowst, agg_sh.at[idxct], add=True)

        def load_idx(ch, b):
            off = pl.multiple_of(ebase + ch * KC, 8)
            pltpu.async_copy(row_hbm.at[pl.ds(off, KC)], idxr.at[b], isems[b])
            pltpu.async_copy(col_hbm.at[pl.ds(off, KC)], idxc.at[b], isems[b])

        def wait_idx(b):
            pltpu.make_async_copy(row_hbm.at[pl.ds(0, KC)], idxr.at[b],
                                  isems[b]).wait()
            pltpu.make_async_copy(col_hbm.at[pl.ds(0, KC)], idxc.at[b],
                                  isems[b]).wait()

        def start_gather(b):
            pltpu.async_copy(gp_hbm.at[idxr.at[b]], rows.at[b], sems[b])

        def wait_gather(b):
            pltpu.make_async_copy(gp_hbm.at[idxr.at[b]], rows.at[b],
                                  sems[b]).wait()

        load_idx(0, 0)
        wait_idx(0)
        start_gather(0)

        def pair(j, _):
            load_idx(2 * j + 1, 1)
            wait_gather(0)
            wait_idx(1)
            start_gather(1)
            pltpu.sync_copy(rows.at[0], agg_sh.at[idxc.at[0]], add=True)
            last = j == NCHM // 2 - 1

            @pl.when(jnp.logical_not(last))
            def _():
                load_idx(2 * j + 2, 0)
            wait_gather(1)

            @pl.when(jnp.logical_not(last))
            def _():
                wait_idx(0)
                start_gather(0)
            pltpu.sync_copy(rows.at[1], agg_sh.at[idxc.at[1]], add=True)
            return 0
        lax.fori_loop(0, NCHM // 2, pair, 0)
        plsc.subcore_barrier()

        @pl.when(s < 15)
        def _():
            for j in range(RS // CH2):
                pltpu.sync_copy(agg_sh.at[pl.ds(s * RS + j * CH2, CH2)],
                                stage)
                pltpu.sync_copy(stage,
                                out_hbm.at[c, pl.ds(s * RS + j * CH2, CH2)])

        @pl.when(s == 15)
        def _():
            for j in range(RSL // CH2L):
                pltpu.sync_copy(agg_sh.at[pl.ds(15 * RS + j * CH2L, CH2L)],
                                stage.at[pl.ds(0, CH2L)])
                pltpu.sync_copy(stage.at[pl.ds(0, CH2L)],
                                out_hbm.at[c, pl.ds(15 * RS + j * CH2L, CH2L)])

    return k(row, col, gp, zeros2d)


def _b_body(degt_ref, g0_ref, s_ref, gp_ref):
    d = jnp.sum(degt_ref[...], axis=1, keepdims=True)
    s = jnp.where(d > 0.0, lax.rsqrt(jnp.maximum(d, 1e-30)), 0.0)
    s_ref[...] = s
    gp_ref[...] = s * g0_ref[...]


def _stage_b(degt, g0):
    row = lambda i: (i, 0)
    return pl.pallas_call(
        _b_body,
        grid=(GRID,),
        in_specs=[pl.BlockSpec((BLK, NC), row), pl.BlockSpec((BLK, HID), row)],
        out_specs=[pl.BlockSpec((BLK, 1), row), pl.BlockSpec((BLK, HID), row)],
        out_shape=[jax.ShapeDtypeStruct((N_NODES, 1), jnp.float32),
                   jax.ShapeDtypeStruct((N_NODES, HID), jnp.float32)],
    )(degt, g0)


def _e_body(q_ref, v_ref, layer0_ref, kv_ref, ksum_ref, sq_ref, sk_ref,
            g1g_ref, g1b_ref, x1_ref):
    den = jnp.sqrt(sq_ref[0, 0]) * jnp.sqrt(sk_ref[0, 0])
    q = q_ref[...]
    v = v_ref[...]
    num = jnp.dot(q, kv_ref[...], preferred_element_type=jnp.float32) / den \
        + N_NODES * v
    nrm = lax.dot_general(q, ksum_ref[...], (((1,), (1,)), ((), ())),
                          preferred_element_type=jnp.float32) / den + N_NODES
    h = ALPHA * (num / nrm) + (1.0 - ALPHA) * layer0_ref[...]
    mu = jnp.mean(h, axis=1, keepdims=True)
    var = jnp.mean((h - mu) ** 2, axis=1, keepdims=True)
    h = g1g_ref[...] * (h - mu) * lax.rsqrt(var + EPS) + g1b_ref[...]
    x1_ref[...] = jnp.maximum(h, 0.0)


def _stage_e(q, v, layer0, kv, ksum, sq, sk, g1g, g1b):
    row = lambda i: (i, 0)
    acc = lambda i: (0, 0)
    return pl.pallas_call(
        _e_body,
        grid=(GRID,),
        in_specs=[pl.BlockSpec((BLK, HID), row)] * 3 + [
            pl.BlockSpec((HID, HID), acc),
            pl.BlockSpec((1, HID), acc),
            pl.BlockSpec((1, HID), acc),
            pl.BlockSpec((1, HID), acc),
            pl.BlockSpec((1, HID), acc),
            pl.BlockSpec((1, HID), acc),
        ],
        out_specs=pl.BlockSpec((BLK, HID), row),
        out_shape=jax.ShapeDtypeStruct((N_NODES, HID), jnp.float32),
    )(q, v, layer0, kv, ksum, sq, sk, g1g, g1b)


def _c_body(pa_ref, pb_ref, s_ref, g0_ref, x1_ref, wc_ref, bc_ref,
            wf_ref, bf_ref, out_ref):
    agg = s_ref[...] * (pa_ref[...] + pb_ref[...])
    g2 = jnp.dot(agg, wc_ref[...], preferred_element_type=jnp.float32) \
        + bc_ref[...]
    g2 = jnp.maximum(g2 * IBN, 0.0)
    x2 = g2 + g0_ref[...]
    comb = GW * x2 + (1.0 - GW) * x1_ref[...]
    out_ref[...] = jnp.dot(comb, wf_ref[...],
                           preferred_element_type=jnp.float32) + bf_ref[...]


def _stage_c(pa, pb, s, g0, x1, wc, bc, wf, bf):
    row = lambda i: (i, 0)
    acc = lambda i: (0, 0)
    return pl.pallas_call(
        _c_body,
        grid=(GRID,),
        in_specs=[pl.BlockSpec((BLK, HID), row), pl.BlockSpec((BLK, HID), row),
                  pl.BlockSpec((BLK, 1), row), pl.BlockSpec((BLK, HID), row),
                  pl.BlockSpec((BLK, HID), row),
                  pl.BlockSpec((HID, HID), acc), pl.BlockSpec((1, HID), acc),
                  pl.BlockSpec((HID, OUT_CH), acc),
                  pl.BlockSpec((1, OUT_CH), acc)],
        out_specs=pl.BlockSpec((BLK, OUT_CH), row),
        out_shape=jax.ShapeDtypeStruct((N_NODES, OUT_CH), jnp.float32),
    )(pa, pb, s, g0, x1, wc, bc, wf, bf)


def kernel(x, edge_index, params):
    p = params
    wts = [p['t_fc_W'].T, p['t_fc_b'][None, :],
           p['t_ln0_g'][None, :], p['t_ln0_b'][None, :],
           p['Wq'].T, p['bq'][None, :], p['Wk'].T, p['bk'][None, :],
           p['Wv'].T, p['bv'][None, :], p['g_fc_W'].T, p['g_fc_b'][None, :]]
    layer0, q, v, g0, kv, ksum, sq, sk = _stage_a(x, wts)

    erow = edge_index[0]
    ecol = edge_index[1]
    degp = _sc_deg(ecol)
    degt = degp.reshape(NC, N_NODES).T
    s, gp = _stage_b(degt, g0)

    zeros2d = jnp.zeros((CH2, HID), jnp.float32)
    aggp = _sc_agg(erow, ecol, gp, zeros2d)

    x1 = _stage_e(q, v, layer0, kv, ksum, sq, sk,
                  p['t_ln1_g'][None, :], p['t_ln1_b'][None, :])

    return _stage_c(aggp[0], aggp[1], s, g0, x1,
                    p['g_conv_W'].T, p['g_conv_b'][None, :],
                    p['fc_W'].T, p['fc_b'][None, :])


# speedup vs baseline: 26.1521x; 1.0261x over previous
import functools

import jax
import jax.numpy as jnp
from jax import lax
from jax.experimental import pallas as pl
from jax.experimental.pallas import tpu as pltpu
from jax.experimental.pallas import tpu_sc as plsc

N_NODES = 10000
N_EDGES = 320000
HID = 128
OUT_CH = 40
ALPHA = 0.5
GW = 0.8
EPS = 1e-5
IBN = 1.0 / (1.0 + EPS) ** 0.5

BLK = 2000
GRID = N_NODES // BLK
NC, NS = 2, 16
NW = NC * NS
EPT = N_EDGES // NW
KC = 128
NCHM = EPT // KC
TAIL = EPT - NCHM * KC


def _a_body(x_ref, wt_ref, bt_ref, g0g_ref, g0b_ref, wq_ref, bq_ref,
            wk_ref, bk_ref, wv_ref, bv_ref, wg_ref, bg_ref,
            layer0_ref, q_ref, v_ref, g0_ref, kv_ref, ksum_ref, sq_ref,
            sk_ref):
    i = pl.program_id(0)
    x = x_ref[...]
    h = jnp.dot(x, wt_ref[...], preferred_element_type=jnp.float32) + bt_ref[...]
    mu = jnp.mean(h, axis=1, keepdims=True)
    var = jnp.mean((h - mu) ** 2, axis=1, keepdims=True)
    h = g0g_ref[...] * (h - mu) * lax.rsqrt(var + EPS) + g0b_ref[...]
    h = jnp.maximum(h, 0.0)
    layer0_ref[...] = h
    q = jnp.dot(h, wq_ref[...], preferred_element_type=jnp.float32) + bq_ref[...]
    k = jnp.dot(h, wk_ref[...], preferred_element_type=jnp.float32) + bk_ref[...]
    v = jnp.dot(h, wv_ref[...], preferred_element_type=jnp.float32) + bv_ref[...]
    q_ref[...] = q
    v_ref[...] = v
    g = jnp.dot(x, wg_ref[...], preferred_element_type=jnp.float32) + bg_ref[...]
    g0_ref[...] = jnp.maximum(g * IBN, 0.0)
    kv = lax.dot_general(k, v, (((0,), (0,)), ((), ())),
                         preferred_element_type=jnp.float32)
    ks = jnp.sum(k, axis=0, keepdims=True)
    sq = jnp.sum(q * q)
    sk = jnp.sum(k * k)

    @pl.when(i == 0)
    def _():
        kv_ref[...] = kv
        ksum_ref[...] = ks
        sq_ref[...] = jnp.full((1, HID), sq, jnp.float32)
        sk_ref[...] = jnp.full((1, HID), sk, jnp.float32)

    @pl.when(i != 0)
    def _():
        kv_ref[...] += kv
        ksum_ref[...] += ks
        sq_ref[...] += sq
        sk_ref[...] += sk


def _stage_a(x, p):
    row = lambda i: (i, 0)
    acc = lambda i: (0, 0)
    outs = jax.ShapeDtypeStruct((N_NODES, HID), jnp.float32)
    return pl.pallas_call(
        _a_body,
        grid=(GRID,),
        in_specs=[pl.BlockSpec((BLK, HID), row)] + [pl.BlockSpec(w.shape, acc)
                                                   for w in p],
        out_specs=[pl.BlockSpec((BLK, HID), row)] * 4 + [
            pl.BlockSpec((HID, HID), acc),
            pl.BlockSpec((1, HID), acc),
            pl.BlockSpec((1, HID), acc),
            pl.BlockSpec((1, HID), acc),
        ],
        out_shape=[outs, outs, outs, outs,
                   jax.ShapeDtypeStruct((HID, HID), jnp.float32),
                   jax.ShapeDtypeStruct((1, HID), jnp.float32),
                   jax.ShapeDtypeStruct((1, HID), jnp.float32),
                   jax.ShapeDtypeStruct((1, HID), jnp.float32)],
    )(x, *p)


RS = 640
RSL = N_NODES - 15 * RS
CH2 = 64
CH2L = 40


def _sc_deg(col):
    mesh = plsc.VectorSubcoreMesh(core_axis_name="c", subcore_axis_name="s")

    @functools.partial(
        pl.kernel, mesh=mesh,
        out_type=jax.ShapeDtypeStruct((NC * N_NODES,), jnp.float32),
        scratch_types=[
            pltpu.VMEM((2, KC), jnp.int32),
            pltpu.VMEM((TAIL,), jnp.int32),
            pltpu.VMEM((KC,), jnp.float32),
            pltpu.VMEM((RS,), jnp.float32),
            pltpu.VMEM_SHARED((N_NODES,), jnp.float32),
            pltpu.SemaphoreType.DMA,
            pltpu.SemaphoreType.DMA,
        ],
    )
    def k(col_hbm, out_hbm, idx2, idxt, ones_v, zbuf, deg_sh, sd0, sd1):
        c = lax.axis_index("c")
        s = lax.axis_index("s")
        sems = (sd0, sd1)

        def fill(j, _):
            ones_v[pl.ds(j * 16, 16)] = jnp.full((16,), 1.0, jnp.float32)
            return 0
        lax.fori_loop(0, KC // 16, fill, 0)

        def zfill(j, _):
            zbuf[pl.ds(j * 16, 16)] = jnp.zeros((16,), jnp.float32)
            return 0
        lax.fori_loop(0, RS // 16, zfill, 0)

        @pl.when(s < 15)
        def _():
            pltpu.sync_copy(zbuf, deg_sh.at[pl.ds(s * RS, RS)])

        @pl.when(s == 15)
        def _():
            pltpu.sync_copy(zbuf.at[pl.ds(0, RSL)],
                            deg_sh.at[pl.ds(15 * RS, RSL)])
        plsc.subcore_barrier()

        ebase = (c * NS + s) * EPT

        toff = pl.multiple_of(ebase + NCHM * KC, 8)
        pltpu.sync_copy(col_hbm.at[pl.ds(toff, TAIL)], idxt)
        pltpu.sync_copy(ones_v.at[pl.ds(0, TAIL)], deg_sh.at[idxt], add=True)

        def load_idx(ch, b):
            off = pl.multiple_of(ebase + ch * KC, 8)
            pltpu.async_copy(col_hbm.at[pl.ds(off, KC)], idx2.at[b], sems[b])

        def wait_idx(b):
            pltpu.make_async_copy(col_hbm.at[pl.ds(0, KC)], idx2.at[b],
                                  sems[b]).wait()

        load_idx(0, 0)

        def pair(j, _):
            wait_idx(0)
            load_idx(2 * j + 1, 1)
            pltpu.sync_copy(ones_v, deg_sh.at[idx2.at[0]], add=True)
            wait_idx(1)

            @pl.when(j != NCHM // 2 - 1)
            def _():
                load_idx(2 * j + 2, 0)
            pltpu.sync_copy(ones_v, deg_sh.at[idx2.at[1]], add=True)
            return 0
        lax.fori_loop(0, NCHM // 2, pair, 0)
        plsc.subcore_barrier()

        obase = c * N_NODES

        @pl.when(s < 15)
        def _():
            pltpu.sync_copy(deg_sh.at[pl.ds(s * RS, RS)], zbuf)
            pltpu.sync_copy(zbuf, out_hbm.at[pl.ds(obase + s * RS, RS)])

        @pl.when(s == 15)
        def _():
            pltpu.sync_copy(deg_sh.at[pl.ds(15 * RS, RSL)],
                            zbuf.at[pl.ds(0, RSL)])
            pltpu.sync_copy(zbuf.at[pl.ds(0, RSL)],
                            out_hbm.at[pl.ds(obase + 15 * RS, RSL)])

    return k(col)


def _sc_agg(row, col, gp, zeros2d):
    mesh = plsc.VectorSubcoreMesh(core_axis_name="c", subcore_axis_name="s")

    @functools.partial(
        pl.kernel, mesh=mesh,
        out_type=jax.ShapeDtypeStruct((NC, N_NODES, HID), jnp.float32),
        scratch_types=[
            pltpu.VMEM((2, KC), jnp.int32),
            pltpu.VMEM((2, KC), jnp.int32),
            pltpu.VMEM((2, KC, HID), jnp.float32),
            pltpu.VMEM((TAIL,), jnp.int32),
            pltpu.VMEM((TAIL,), jnp.int32),
            pltpu.VMEM((TAIL, HID), jnp.float32),
            pltpu.VMEM((CH2, HID), jnp.float32),
            pltpu.VMEM_SHARED((N_NODES, HID), jnp.float32),
            pltpu.SemaphoreType.DMA,
            pltpu.SemaphoreType.DMA,
            pltpu.SemaphoreType.DMA,
            pltpu.SemaphoreType.DMA,
        ],
    )
    def k(row_hbm, col_hbm, gp_hbm, z_hbm, out_hbm, idxr, idxc, rows,
          idxrt, idxct, rowst, stage, agg_sh, sg0, sg1, si0, si1):
        c = lax.axis_index("c")
        s = lax.axis_index("s")
        sems = (sg0, sg1)
        isems = (si0, si1)
        pltpu.sync_copy(z_hbm, stage)

        @pl.when(s < 15)
        def _():
            for j in range(RS // CH2):
                pltpu.sync_copy(stage,
                                agg_sh.at[pl.ds(s * RS + j * CH2, CH2)])

        @pl.when(s == 15)
        def _():
            for j in range(RSL // CH2L):
                pltpu.sync_copy(stage.at[pl.ds(0, CH2L)],
                                agg_sh.at[pl.ds(15 * RS + j * CH2L, CH2L)])
        plsc.subcore_barrier()

        ebase = (c * NS + s) * EPT

        toff = pl.multiple_of(ebase + NCHM * KC, 8)
        pltpu.sync_copy(row_hbm.at[pl.ds(toff, TAIL)], idxrt)
        pltpu.sync_copy(col_hbm.at[pl.ds(toff, TAIL)], idxct)
        pltpu.async_copy(gp_hbm.at[idxrt], rowst, sg0).wait()
        pltpu.sync_copy(rowst, agg_sh.at[idxct], add=True)

        def load_idx(ch, b):
            off = pl.multiple_of(ebase + ch * KC, 8)
            pltpu.async_copy(row_hbm.at[pl.ds(off, KC)], idxr.at[b], isems[b])
            pltpu.async_copy(col_hbm.at[pl.ds(off, KC)], idxc.at[b], isems[b])

        def wait_idx(b):
            pltpu.make_async_copy(row_hbm.at[pl.ds(0, KC)], idxr.at[b],
                                  isems[b]).wait()
            pltpu.make_async_copy(col_hbm.at[pl.ds(0, KC)], idxc.at[b],
                                  isems[b]).wait()

        def start_gather(b):
            pltpu.async_copy(gp_hbm.at[idxr.at[b]], rows.at[b], sems[b])

        def wait_gather(b):
            pltpu.make_async_copy(gp_hbm.at[idxr.at[b]], rows.at[b],
                                  sems[b]).wait()

        load_idx(0, 0)
        wait_idx(0)
        start_gather(0)

        def pair(j, _):
            load_idx(2 * j + 1, 1)
            wait_gather(0)
            wait_idx(1)
            start_gather(1)
            pltpu.sync_copy(rows.at[0], agg_sh.at[idxc.at[0]], add=True)
            last = j == NCHM // 2 - 1

            @pl.when(jnp.logical_not(last))
            def _():
                load_idx(2 * j + 2, 0)
            wait_gather(1)

            @pl.when(jnp.logical_not(last))
            def _():
                wait_idx(0)
                start_gather(0)
            pltpu.sync_copy(rows.at[1], agg_sh.at[idxc.at[1]], add=True)
            return 0
        lax.fori_loop(0, NCHM // 2, pair, 0)
        plsc.subcore_barrier()

        @pl.when(s < 15)
        def _():
            for j in range(RS // CH2):
                pltpu.sync_copy(agg_sh.at[pl.ds(s * RS + j * CH2, CH2)],
                                stage)
                pltpu.sync_copy(stage,
                                out_hbm.at[c, pl.ds(s * RS + j * CH2, CH2)])

        @pl.when(s == 15)
        def _():
            for j in range(RSL // CH2L):
                pltpu.sync_copy(agg_sh.at[pl.ds(15 * RS + j * CH2L, CH2L)],
                                stage.at[pl.ds(0, CH2L)])
                pltpu.sync_copy(stage.at[pl.ds(0, CH2L)],
                                out_hbm.at[c, pl.ds(15 * RS + j * CH2L, CH2L)])

    return k(row, col, gp, zeros2d)


def _b_body(degt_ref, g0_ref, s_ref, gp_ref):
    d = jnp.sum(degt_ref[...], axis=1, keepdims=True)
    s = jnp.where(d > 0.0, lax.rsqrt(jnp.maximum(d, 1e-30)), 0.0)
    s_ref[...] = s
    gp_ref[...] = s * g0_ref[...]


def _stage_b(degt, g0):
    row = lambda i: (i, 0)
    return pl.pallas_call(
        _b_body,
        grid=(GRID,),
        in_specs=[pl.BlockSpec((BLK, NC), row), pl.BlockSpec((BLK, HID), row)],
        out_specs=[pl.BlockSpec((BLK, 1), row), pl.BlockSpec((BLK, HID), row)],
        out_shape=[jax.ShapeDtypeStruct((N_NODES, 1), jnp.float32),
                   jax.ShapeDtypeStruct((N_NODES, HID), jnp.float32)],
    )(degt, g0)


def _c_body(pa_ref, pb_ref, s_ref, g0_ref, q_ref, v_ref, l0_ref, kv_ref,
            ksum_ref, sq_ref, sk_ref, g1g_ref, g1b_ref, wc_ref, bc_ref,
            wf_ref, bf_ref, out_ref):
    den = jnp.sqrt(sq_ref[0, 0]) * jnp.sqrt(sk_ref[0, 0])
    q = q_ref[...]
    v = v_ref[...]
    num = jnp.dot(q, kv_ref[...], preferred_element_type=jnp.float32) / den \
        + N_NODES * v
    nrm = lax.dot_general(q, ksum_ref[...], (((1,), (1,)), ((), ())),
                          preferred_element_type=jnp.float32) / den + N_NODES
    h = ALPHA * (num / nrm) + (1.0 - ALPHA) * l0_ref[...]
    mu = jnp.mean(h, axis=1, keepdims=True)
    var = jnp.mean((h - mu) ** 2, axis=1, keepdims=True)
    h = g1g_ref[...] * (h - mu) * lax.rsqrt(var + EPS) + g1b_ref[...]
    x1 = jnp.maximum(h, 0.0)
    agg = s_ref[...] * (pa_ref[0] + pb_ref[0])
    g2 = jnp.dot(agg, wc_ref[...], preferred_element_type=jnp.float32) \
        + bc_ref[...]
    g2 = jnp.maximum(g2 * IBN, 0.0)
    x2 = g2 + g0_ref[...]
    comb = GW * x2 + (1.0 - GW) * x1
    out_ref[...] = jnp.dot(comb, wf_ref[...],
                           preferred_element_type=jnp.float32) + bf_ref[...]


def _stage_c(pa, pb, s, g0, q, v, l0, kv, ksum, sq, sk, g1g, g1b,
             wc, bc, wf, bf):
    row = lambda i: (i, 0)
    acc = lambda i: (0, 0)
    return pl.pallas_call(
        _c_body,
        grid=(GRID,),
        in_specs=[pl.BlockSpec((1, BLK, HID), lambda i: (0, i, 0)),
                  pl.BlockSpec((1, BLK, HID), lambda i: (1, i, 0)),
                  pl.BlockSpec((BLK, 1), row), pl.BlockSpec((BLK, HID), row),
                  pl.BlockSpec((BLK, HID), row), pl.BlockSpec((BLK, HID), row),
                  pl.BlockSpec((BLK, HID), row),
                  pl.BlockSpec((HID, HID), acc),
                  pl.BlockSpec((1, HID), acc), pl.BlockSpec((1, HID), acc),
                  pl.BlockSpec((1, HID), acc), pl.BlockSpec((1, HID), acc),
                  pl.BlockSpec((1, HID), acc),
                  pl.BlockSpec((HID, HID), acc), pl.BlockSpec((1, HID), acc),
                  pl.BlockSpec((HID, OUT_CH), acc),
                  pl.BlockSpec((1, OUT_CH), acc)],
        out_specs=pl.BlockSpec((BLK, OUT_CH), row),
        out_shape=jax.ShapeDtypeStruct((N_NODES, OUT_CH), jnp.float32),
    )(pa, pb, s, g0, q, v, l0, kv, ksum, sq, sk, g1g, g1b, wc, bc, wf, bf)


def kernel(x, edge_index, params):
    p = params
    wts = [p['t_fc_W'].T, p['t_fc_b'][None, :],
           p['t_ln0_g'][None, :], p['t_ln0_b'][None, :],
           p['Wq'].T, p['bq'][None, :], p['Wk'].T, p['bk'][None, :],
           p['Wv'].T, p['bv'][None, :], p['g_fc_W'].T, p['g_fc_b'][None, :]]
    layer0, q, v, g0, kv, ksum, sq, sk = _stage_a(x, wts)

    erow = edge_index[0]
    ecol = edge_index[1]
    degp = _sc_deg(ecol)
    degt = degp.reshape(NC, N_NODES).T
    s, gp = _stage_b(degt, g0)

    zeros2d = jnp.zeros((CH2, HID), jnp.float32)
    aggp = _sc_agg(erow, ecol, gp, zeros2d)

    return _stage_c(aggp, aggp, s, g0, q, v, layer0, kv, ksum, sq, sk,
                    p['t_ln1_g'][None, :], p['t_ln1_b'][None, :],
                    p['g_conv_W'].T, p['g_conv_b'][None, :],
                    p['fc_W'].T, p['fc_b'][None, :])
